# baseline (speedup 1.0000x reference)
import jax
import jax.numpy as jnp
from jax.experimental import pallas as pl
from jax.experimental.pallas import tpu as pltpu

_NBUF = 4
_BR = 200


def _gcn_kernel(adj_hbm, x_ref, w_ref, b_ref, out_ref, s_ref, bufs, sems):
    n = x_ref.shape[0]
    nsteps = n // _BR

    for k in range(_NBUF):
        pltpu.make_async_copy(
            adj_hbm.at[pl.ds(k * _BR, _BR), :], bufs.at[k], sems.at[k]
        ).start()

    s_ref[...] = jnp.dot(x_ref[...], w_ref[...],
                         preferred_element_type=jnp.float32)

    def body(i, carry):
        slot = jax.lax.rem(i, _NBUF)
        pltpu.make_async_copy(
            adj_hbm.at[pl.ds(i * _BR, _BR), :], bufs.at[slot], sems.at[slot]
        ).wait()
        out_ref[pl.ds(i * _BR, _BR), :] = jnp.dot(
            bufs[slot], s_ref[...],
            preferred_element_type=jnp.float32) + b_ref[...]

        nxt = i + _NBUF

        @pl.when(nxt < nsteps)
        def _():
            pltpu.make_async_copy(
                adj_hbm.at[pl.ds(nxt * _BR, _BR), :], bufs.at[slot],
                sems.at[slot]
            ).start()

        return carry

    jax.lax.fori_loop(0, nsteps, body, 0)


def kernel(x, adj, W, b):
    n, d_in = x.shape
    d_out = W.shape[1]

    out = pl.pallas_call(
        _gcn_kernel,
        grid=(1,),
        in_specs=[
            pl.BlockSpec(memory_space=pltpu.MemorySpace.HBM),
            pl.BlockSpec((n, d_in), lambda i: (0, 0)),
            pl.BlockSpec((d_in, d_out), lambda i: (0, 0)),
            pl.BlockSpec((1, d_out), lambda i: (0, 0)),
        ],
        out_specs=pl.BlockSpec((n, d_out), lambda i: (0, 0)),
        out_shape=jax.ShapeDtypeStruct((n, d_out), jnp.float32),
        scratch_shapes=[
            pltpu.VMEM((n, d_out), jnp.float32),
            pltpu.VMEM((_NBUF, _BR, n), jnp.float32),
            pltpu.SemaphoreType.DMA((_NBUF,)),
        ],
    )(adj, x, W, b.reshape(1, d_out))
    return out


# speedup vs baseline: 1.0190x; 1.0190x over previous
import jax
import jax.numpy as jnp
from jax.experimental import pallas as pl
from jax.experimental.pallas import tpu as pltpu


def _gcn_kernel(adj_ref, x_ref, w_ref, b_ref, out_ref, s_ref):
    i = pl.program_id(0)

    @pl.when(i == 0)
    def _():
        s_ref[...] = jnp.dot(x_ref[...], w_ref[...],
                             preferred_element_type=jnp.float32)

    out_ref[...] = jnp.dot(adj_ref[...], s_ref[...],
                           preferred_element_type=jnp.float32) + b_ref[...]


def kernel(x, adj, W, b):
    n, d_in = x.shape
    d_out = W.shape[1]

    br = 400
    out = pl.pallas_call(
        _gcn_kernel,
        grid=(n // br,),
        in_specs=[
            pl.BlockSpec((br, n), lambda i: (i, 0)),
            pl.BlockSpec((n, d_in), lambda i: (0, 0)),
            pl.BlockSpec((d_in, d_out), lambda i: (0, 0)),
            pl.BlockSpec((1, d_out), lambda i: (0, 0)),
        ],
        out_specs=pl.BlockSpec((br, d_out), lambda i: (i, 0)),
        out_shape=jax.ShapeDtypeStruct((n, d_out), jnp.float32),
        scratch_shapes=[pltpu.VMEM((n, d_out), jnp.float32)],
    )(adj, x, W, b.reshape(1, d_out))
    return out
